# Initial kernel scaffold; baseline (speedup 1.0000x reference)
import jax
import jax.numpy as jnp
from jax.experimental import pallas as pl


def kernel(input_batch, table):
    raise NotImplementedError("write your pallas kernel here")


# speedup vs baseline: 1.2529x; 1.2529x over previous
import functools

import jax
import jax.numpy as jnp
from jax import lax
from jax.experimental import pallas as pl
from jax.experimental.pallas import tpu as pltpu
from jax.experimental.pallas import tpu_sc as plsc

_VOCAB = 100000
_D = 128
_B = 4096
_F = 26
_N = _B * _F
_NC = 2
_NS = 16
_NW = _NC * _NS
_PER_W = _N // _NW
_CHUNK = 128
_NCHUNK = _PER_W // _CHUNK


def _make_gather():
  mesh = plsc.VectorSubcoreMesh(core_axis_name="c", subcore_axis_name="s")

  @functools.partial(
      pl.kernel,
      mesh=mesh,
      out_type=jax.ShapeDtypeStruct((_N, _D), jnp.float32),
      scratch_types=[
          pltpu.VMEM((_NCHUNK, _CHUNK), jnp.int32),
          pltpu.VMEM((_CHUNK, _D), jnp.float32),
          pltpu.VMEM((_CHUNK, _D), jnp.float32),
          pltpu.SemaphoreType.DMA,
          pltpu.SemaphoreType.DMA,
          pltpu.SemaphoreType.DMA,
          pltpu.SemaphoreType.DMA,
      ],
  )
  def gather_kernel(table_hbm, idx_hbm, out_hbm, idx_v, buf0, buf1,
                    g0, g1, w0, w1):
    wid = lax.axis_index("s") * _NC + lax.axis_index("c")
    base = wid * _PER_W
    pltpu.sync_copy(idx_hbm.at[wid], idx_v)

    def body(j, carry):
      c0 = 2 * j
      c1 = c0 + 1
      cp0 = pltpu.async_copy(table_hbm.at[idx_v.at[c0]], buf0, g0)
      cp1 = pltpu.async_copy(table_hbm.at[idx_v.at[c1]], buf1, g1)
      cp0.wait()
      wb0 = pltpu.async_copy(
          buf0, out_hbm.at[pl.ds(base + c0 * _CHUNK, _CHUNK)], w0)
      cp1.wait()
      wb1 = pltpu.async_copy(
          buf1, out_hbm.at[pl.ds(base + c1 * _CHUNK, _CHUNK)], w1)
      wb0.wait()
      wb1.wait()
      return carry

    lax.fori_loop(0, _NCHUNK // 2, body, 0)

  return gather_kernel


_gather = _make_gather()


def kernel(input_batch, table):
  idx3 = input_batch.reshape(_NW, _NCHUNK, _CHUNK)
  out = _gather(table, idx3)
  return out.reshape(_B, _F, _D)


# speedup vs baseline: 1.2921x; 1.0314x over previous
import functools

import jax
import jax.numpy as jnp
from jax import lax
from jax.experimental import pallas as pl
from jax.experimental.pallas import tpu as pltpu
from jax.experimental.pallas import tpu_sc as plsc

_VOCAB = 100000
_D = 128
_B = 4096
_F = 26
_N = _B * _F
_NC = 2
_NS = 16
_NW = _NC * _NS
_PER_W = _N // _NW
_CHUNK = 128
_NCHUNK = _PER_W // _CHUNK
_STEP = 2 * _CHUNK
_NSTEP = _NCHUNK // 2


def _make_gather():
  mesh = plsc.VectorSubcoreMesh(core_axis_name="c", subcore_axis_name="s")

  @functools.partial(
      pl.kernel,
      mesh=mesh,
      out_type=jax.ShapeDtypeStruct((_N, _D), jnp.float32),
      scratch_types=[
          pltpu.VMEM((_NCHUNK, _CHUNK), jnp.int32),
          pltpu.VMEM((_STEP, _D), jnp.float32),
          pltpu.VMEM((_STEP, _D), jnp.float32),
          pltpu.SemaphoreType.DMA,
          pltpu.SemaphoreType.DMA,
          pltpu.SemaphoreType.DMA,
          pltpu.SemaphoreType.DMA,
      ],
  )
  def gather_kernel(table_hbm, idx_hbm, out_hbm, idx_v, buf_a, buf_b,
                    ga, gb, wa, wb):
    wid = lax.axis_index("s") * _NC + lax.axis_index("c")
    base = wid * _PER_W
    pltpu.sync_copy(idx_hbm.at[wid], idx_v)

    def g_start(step, buf, sem):
      c0 = 2 * step
      pltpu.async_copy(
          table_hbm.at[idx_v.at[c0]], buf.at[pl.ds(0, _CHUNK)], sem)
      pltpu.async_copy(
          table_hbm.at[idx_v.at[c0 + 1]], buf.at[pl.ds(_CHUNK, _CHUNK)], sem)

    def g_wait(step, buf, sem):
      c0 = 2 * step
      pltpu.make_async_copy(
          table_hbm.at[idx_v.at[c0]], buf.at[pl.ds(0, _CHUNK)], sem).wait()
      pltpu.make_async_copy(
          table_hbm.at[idx_v.at[c0 + 1]], buf.at[pl.ds(_CHUNK, _CHUNK)],
          sem).wait()

    def w_start(step, buf, sem):
      pltpu.async_copy(
          buf, out_hbm.at[pl.ds(base + step * _STEP, _STEP)], sem)

    def w_wait(step, buf, sem):
      pltpu.make_async_copy(
          buf, out_hbm.at[pl.ds(base + step * _STEP, _STEP)], sem).wait()

    g_start(0, buf_a, ga)
    g_start(1, buf_b, gb)
    g_wait(0, buf_a, ga)
    w_start(0, buf_a, wa)

    def body(i, carry):
      j = 2 * i + 1
      w_wait(j - 1, buf_a, wa)
      g_start(j + 1, buf_a, ga)
      g_wait(j, buf_b, gb)
      w_start(j, buf_b, wb)
      w_wait(j, buf_b, wb)
      g_start(j + 2, buf_b, gb)
      g_wait(j + 1, buf_a, ga)
      w_start(j + 1, buf_a, wa)
      return carry

    lax.fori_loop(0, (_NSTEP - 3) // 2, body, 0)

    w_wait(10, buf_a, wa)
    g_start(12, buf_a, ga)
    g_wait(11, buf_b, gb)
    w_start(11, buf_b, wb)
    w_wait(11, buf_b, wb)
    g_wait(12, buf_a, ga)
    w_start(12, buf_a, wa)
    w_wait(12, buf_a, wa)

  return gather_kernel


_gather = _make_gather()


def kernel(input_batch, table):
  idx3 = input_batch.reshape(_NW, _NCHUNK, _CHUNK)
  out = _gather(table, idx3)
  return out.reshape(_B, _F, _D)


# speedup vs baseline: 3.6558x; 2.8293x over previous
import functools

import jax
import jax.numpy as jnp
from jax import lax
from jax.experimental import pallas as pl
from jax.experimental.pallas import tpu as pltpu
from jax.experimental.pallas import tpu_sc as plsc

_VOCAB = 100000
_D = 128
_B = 4096
_F = 26
_N = _B * _F
_NC = 2
_NS = 16
_NW = _NC * _NS
_PER_W = _N // _NW
_CHUNK = 128
_NCHUNK = _PER_W // _CHUNK
_STEP = 2 * _CHUNK
_NSTEP = _NCHUNK // 2


def _make_gather():
  mesh = plsc.VectorSubcoreMesh(core_axis_name="c", subcore_axis_name="s")

  @functools.partial(
      pl.kernel,
      mesh=mesh,
      out_type=jax.ShapeDtypeStruct((_N, _D), jnp.float32),
      scratch_types=[
          pltpu.VMEM((_NCHUNK, _CHUNK), jnp.int32),
          pltpu.VMEM((_STEP, _D), jnp.float32),
          pltpu.VMEM((_STEP, _D), jnp.float32),
          pltpu.SemaphoreType.DMA,
          pltpu.SemaphoreType.DMA,
          pltpu.SemaphoreType.DMA,
          pltpu.SemaphoreType.DMA,
      ],
  )
  def gather_kernel(table_hbm, idx_hbm, out_hbm, idx_v, buf_a, buf_b,
                    ga, gb, wa, wb):
    wid = lax.axis_index("s") * _NC + lax.axis_index("c")
    base = wid * _PER_W
    pltpu.sync_copy(idx_hbm.at[wid], idx_v)

    def g_start(step, buf, sem):
      c0 = 2 * step
      pltpu.async_copy(
          table_hbm.at[idx_v.at[c0]], buf.at[pl.ds(0, _CHUNK)], sem)
      pltpu.async_copy(
          table_hbm.at[idx_v.at[c0 + 1]], buf.at[pl.ds(_CHUNK, _CHUNK)], sem)

    def g_wait(step, buf, sem):
      c0 = 2 * step
      pltpu.make_async_copy(
          table_hbm.at[idx_v.at[c0]], buf.at[pl.ds(0, _CHUNK)], sem).wait()
      pltpu.make_async_copy(
          table_hbm.at[idx_v.at[c0 + 1]], buf.at[pl.ds(_CHUNK, _CHUNK)],
          sem).wait()

    def w_start(step, buf, sem):
      pltpu.async_copy(
          buf, out_hbm.at[pl.ds(base + step * _STEP, _STEP)], sem)

    def w_wait(step, buf, sem):
      pltpu.make_async_copy(
          buf, out_hbm.at[pl.ds(base + step * _STEP, _STEP)], sem).wait()

    g_start(0, buf_a, ga)
    g_start(1, buf_b, gb)
    g_wait(0, buf_a, ga)
    w_start(0, buf_a, wa)

    def body(i, carry):
      j = 2 * i + 1
      w_wait(j - 1, buf_a, wa)
      g_start(j + 1, buf_a, ga)
      g_wait(j, buf_b, gb)
      w_start(j, buf_b, wb)
      w_wait(j, buf_b, wb)
      g_start(j + 2, buf_b, gb)
      g_wait(j + 1, buf_a, ga)
      w_start(j + 1, buf_a, wa)
      return carry

    lax.fori_loop(0, (_NSTEP - 3) // 2, body, 0)

    w_wait(10, buf_a, wa)
    g_start(12, buf_a, ga)
    g_wait(11, buf_b, gb)
    w_start(11, buf_b, wb)
    w_wait(11, buf_b, wb)
    g_wait(12, buf_a, ga)
    w_start(12, buf_a, wa)
    w_wait(12, buf_a, wa)

  return gather_kernel


_gather = _make_gather()


def kernel(input_batch, table):
  idx3 = input_batch.T.reshape(_NW, _NCHUNK, _CHUNK)
  out = _gather(table, idx3)
  return out.reshape(_F, _B, _D).transpose(1, 0, 2)
